# baseline (speedup 1.0000x reference)
import functools

import jax
import jax.numpy as jnp
from jax import lax
from jax.experimental import pallas as pl
from jax.experimental.pallas import tpu as pltpu
from jax.experimental.pallas import tpu_sc as plsc

VX, VY = 0.512, 0.512
X0, Y0 = -51.2, -51.2
BEV_H, BEV_W = 200, 200
NCELL = 2 * BEV_H * BEV_W
NPTS = 2 * 100000
SENTINEL = 1 << 20

NW = 32
NGROUP = 128
NPASS = NGROUP // NW
LCELLS = NCELL // NGROUP
LPAD = 640
G = 256
BS = 2000
NBLK = NPTS // BS
NV = BS // 16


def _scatter_max_sc(ids, feats):
    mesh = plsc.VectorSubcoreMesh(core_axis_name="c", subcore_axis_name="s")

    @functools.partial(
        pl.kernel,
        mesh=mesh,
        compiler_params=pltpu.CompilerParams(needs_layout_passes=False),
        out_type=jax.ShapeDtypeStruct((LPAD * NGROUP, 128), jnp.float32),
        scratch_types=[
            pltpu.VMEM((LPAD, 128), jnp.float32),
            pltpu.VMEM((BS,), jnp.int32),
            pltpu.VMEM((BS,), jnp.int32),
            pltpu.VMEM((G + 16,), jnp.int32),
            pltpu.VMEM((G + 16,), jnp.int32),
            pltpu.VMEM((G, 128), jnp.float32),
            pltpu.VMEM((LPAD // 128, 128), jnp.int32),
            pltpu.SemaphoreType.DMA,
            pltpu.SemaphoreType.DMA,
        ],
    )
    def k(ids_hbm, feats_hbm, out_hbm, grid_v, ids_a, ids_b, pidx_v, lcell_v,
          rows_v, widx_v, sem, sem2):
        wid = lax.axis_index("s") * 2 + lax.axis_index("c")
        iota = lax.iota(jnp.int32, 16)
        zf = jnp.zeros((16,), jnp.float32)

        def fire():
            pltpu.async_copy(feats_hbm.at[pidx_v.at[pl.ds(0, G)]],
                             rows_v, sem).wait()

            def rmw16(j, _):
                lvec = lcell_v[pl.ds(j * 16, 16)]
                for t in range(16):
                    l = lvec[t]
                    for r in range(8):
                        cur = grid_v[l, pl.ds(r * 16, 16)]
                        val = rows_v[j * 16 + t, pl.ds(r * 16, 16)]
                        grid_v[l, pl.ds(r * 16, 16)] = jnp.maximum(cur, val)
                return 0

            lax.fori_loop(0, G // 16, rmw16, 0)

        def pass_body(p, _):
            gid = p * NW + wid

            def z_body(i, __):
                for r in range(8):
                    grid_v[i, pl.ds(r * 16, 16)] = zf
                return 0
            lax.fori_loop(0, LPAD, z_body, 0)
            for j in range((G + 16) // 16):
                pidx_v[pl.ds(j * 16, 16)] = iota * 0
                lcell_v[pl.ds(j * 16, 16)] = iota * 0 + (LPAD - 1)

            def scan_block(buf, blk, fill):
                def v_body(v, fill):
                    ids16 = buf[pl.ds(v * 16, 16)]
                    match = (ids16 < NCELL) & ((ids16 & (NGROUP - 1)) == gid)
                    pc = plsc.all_reduce_population_count(match)[0]
                    fill_new = fill + pc

                    @pl.when(pc > 0)
                    def _():
                        csum = plsc.cumsum(match.astype(jnp.int32))
                        pos = fill + csum - 1
                        plsc.store_scatter(lcell_v, [pos],
                                           lax.shift_right_logical(ids16, 7),
                                           mask=match)
                        plsc.store_scatter(pidx_v, [pos],
                                           blk * BS + v * 16 + iota,
                                           mask=match)

                        @pl.when(fill_new >= G)
                        def _():
                            fire()
                            pidx_v[pl.ds(0, 16)] = pidx_v[pl.ds(G, 16)]
                            lcell_v[pl.ds(0, 16)] = lcell_v[pl.ds(G, 16)]

                    return jnp.where(fill_new >= G, fill_new - G, fill_new)

                return lax.fori_loop(0, NV, v_body, fill)

            pltpu.async_copy(ids_hbm.at[pl.ds(0, BS)], ids_a, sem2)

            def blk_body(i, fill):
                blk = 2 * i
                pltpu.make_async_copy(ids_hbm.at[pl.ds(0, BS)], ids_a,
                                      sem2).wait()
                pltpu.async_copy(ids_hbm.at[pl.ds((blk + 1) * BS, BS)],
                                 ids_b, sem2)
                fill = scan_block(ids_a, blk, fill)
                pltpu.make_async_copy(ids_hbm.at[pl.ds(0, BS)], ids_b,
                                      sem2).wait()

                @pl.when(blk + 2 < NBLK)
                def _():
                    pltpu.async_copy(ids_hbm.at[pl.ds((blk + 2) * BS, BS)],
                                     ids_a, sem2)

                return scan_block(ids_b, blk + 1, fill)

            fill = lax.fori_loop(0, NBLK // 2, blk_body, 0)

            @pl.when(fill > 0)
            def _():
                for j in range(G // 16):
                    sl = pl.ds(j * 16, 16)
                    tail = (j * 16 + iota) >= fill
                    lcell_v[sl] = jnp.where(tail, LPAD - 1, lcell_v[sl])
                    pidx_v[sl] = jnp.where(tail, 0, pidx_v[sl])
                fire()

            for kk in range(LPAD // 128):
                for v in range(8):
                    l16 = kk * 128 + v * 16 + iota
                    widx_v[kk, pl.ds(v * 16, 16)] = l16 * NGROUP + gid
            for kk in range(LPAD // 128):
                pltpu.async_copy(grid_v.at[pl.ds(kk * 128, 128), :],
                                 out_hbm.at[widx_v.at[kk]], sem).wait()
            return 0

        lax.fori_loop(0, NPASS, pass_body, 0)

    return k(ids, feats)


def _bn_relu_kernel(y_ref, mean_ref, rstd_ref, g_ref, be_ref, o_ref):
    mean = mean_ref[0, 0, :]
    rstd = rstd_ref[0, 0, :]
    g = g_ref[0, 0, :]
    be = be_ref[0, 0, :]
    y = y_ref[...]
    yn = (y - mean[None, :, None, None]) * rstd[None, :, None, None]
    yn = yn * g[None, :, None, None] + be[None, :, None, None]
    o_ref[...] = jnp.maximum(yn, 0.0)


def _bn_relu(y, g, be):
    B, C, H, W = y.shape
    mean = jnp.mean(y, axis=(0, 2, 3))
    var = jnp.mean((y - mean[None, :, None, None]) ** 2, axis=(0, 2, 3))
    rstd = jax.lax.rsqrt(var + 1e-5)
    cb = 32
    out = pl.pallas_call(
        _bn_relu_kernel,
        grid=(B, C // cb),
        in_specs=[
            pl.BlockSpec((1, cb, H, W), lambda b, c: (b, c, 0, 0)),
            pl.BlockSpec((1, 1, cb), lambda b, c: (c, 0, 0)),
            pl.BlockSpec((1, 1, cb), lambda b, c: (c, 0, 0)),
            pl.BlockSpec((1, 1, cb), lambda b, c: (c, 0, 0)),
            pl.BlockSpec((1, 1, cb), lambda b, c: (c, 0, 0)),
        ],
        out_specs=pl.BlockSpec((1, cb, H, W), lambda b, c: (b, c, 0, 0)),
        out_shape=jax.ShapeDtypeStruct((B, C, H, W), jnp.float32),
    )(y, mean.reshape(C // cb, 1, cb), rstd.reshape(C // cb, 1, cb),
      g.reshape(C // cb, 1, cb), be.reshape(C // cb, 1, cb))
    return out


def _conv_bn_relu(x, w, b, g, be):
    y = jax.lax.conv_general_dilated(
        x, w, (1, 1), 'SAME', dimension_numbers=('NCHW', 'OIHW', 'NCHW'))
    y = y + b[None, :, None, None]
    return _bn_relu(y, g, be)


def kernel(points, W1, b1, g1, be1, W2, b2, c1w, c1b, bn1g, bn1b, c2w, c2b,
           bn2g, bn2b, c3w, c3b, bn3g, bn3b):
    B, N, _ = points.shape
    px = points[..., 0]
    py = points[..., 1]
    nz = (px != 0) | (py != 0)
    gx = ((px - X0) / VX).astype(jnp.int32)
    gy = ((py - Y0) / VY).astype(jnp.int32)
    valid = nz & (gx >= 0) & (gx < BEV_W) & (gy >= 0) & (gy < BEV_H)
    m = valid.astype(jnp.float32)[..., None]
    f = jnp.einsum('bnd,cd->bnc', points, W1) + b1
    cnt = jnp.maximum(jnp.sum(m, axis=1, keepdims=True), 1.0)
    mean = jnp.sum(f * m, axis=1, keepdims=True) / cnt
    var = jnp.sum(((f - mean) ** 2) * m, axis=1, keepdims=True) / cnt
    f = (f - mean) / jnp.sqrt(var + 1e-5) * g1 + be1
    f = jax.nn.relu(f)
    f = jnp.einsum('bnc,oc->bno', f, W2) + b2
    bidx = jnp.broadcast_to(jnp.arange(B)[:, None], (B, N))
    flat = bidx * (BEV_H * BEV_W) + gy * BEV_W + gx
    flat = jnp.where(valid, flat, SENTINEL)

    grid = _scatter_max_sc(flat.reshape(-1), f.reshape(-1, 128))
    grid = grid[:NCELL].reshape(B, BEV_H, BEV_W, 128).transpose(0, 3, 1, 2)

    h = _conv_bn_relu(grid, c1w, c1b, bn1g, bn1b)
    h = _conv_bn_relu(h, c2w, c2b, bn2g, bn2b)
    h = _conv_bn_relu(h, c3w, c3b, bn3g, bn3b)
    return h


# speedup vs baseline: 1.2298x; 1.2298x over previous
import functools

import jax
import jax.numpy as jnp
from jax import lax
from jax.experimental import pallas as pl
from jax.experimental.pallas import tpu as pltpu
from jax.experimental.pallas import tpu_sc as plsc

VX, VY = 0.512, 0.512
X0, Y0 = -51.2, -51.2
BEV_H, BEV_W = 200, 200
NCELL = 2 * BEV_H * BEV_W
NPTS = 2 * 100000
SENTINEL = 1 << 20

NW = 32
NGROUP = 128
NPASS = NGROUP // NW
LCELLS = NCELL // NGROUP
LPAD = 640
G = 256
BS = 2000
NBLK = NPTS // BS
NV = BS // 16


def _scatter_max_sc(ids, feats):
    mesh = plsc.VectorSubcoreMesh(core_axis_name="c", subcore_axis_name="s")

    @functools.partial(
        pl.kernel,
        mesh=mesh,
        compiler_params=pltpu.CompilerParams(needs_layout_passes=False),
        out_type=jax.ShapeDtypeStruct((LPAD * NGROUP, 128), jnp.float32),
        scratch_types=[
            pltpu.VMEM((LPAD, 128), jnp.float32),
            pltpu.VMEM((BS,), jnp.int32),
            pltpu.VMEM((BS,), jnp.int32),
            pltpu.VMEM((G + BS + 48,), jnp.int32),
            pltpu.VMEM((G + BS + 48,), jnp.int32),
            pltpu.VMEM((G, 128), jnp.float32),
            pltpu.VMEM((LPAD // 128, 128), jnp.int32),
            pltpu.SemaphoreType.DMA,
            pltpu.SemaphoreType.DMA,
        ],
    )
    def k(ids_hbm, feats_hbm, out_hbm, grid_v, ids_a, ids_b, pidx_v, lcell_v,
          rows_v, widx_v, sem, sem2):
        wid = lax.axis_index("s") * 2 + lax.axis_index("c")
        iota = lax.iota(jnp.int32, 16)
        zf = jnp.zeros((16,), jnp.float32)

        def fire(off):
            pltpu.async_copy(feats_hbm.at[pidx_v.at[pl.ds(off, G)]],
                             rows_v, sem).wait()

            def rmw16(j, _):
                lvec = lcell_v[pl.ds(off + j * 16, 16)]
                for t in range(16):
                    l = lvec[t]
                    for r in range(8):
                        cur = grid_v[l, pl.ds(r * 16, 16)]
                        val = rows_v[j * 16 + t, pl.ds(r * 16, 16)]
                        grid_v[l, pl.ds(r * 16, 16)] = jnp.maximum(cur, val)
                return 0

            lax.fori_loop(0, G // 16, rmw16, 0)

        def pass_body(p, _):
            gid = p * NW + wid

            def z_body(i, __):
                for r in range(8):
                    grid_v[i, pl.ds(r * 16, 16)] = zf
                return 0
            lax.fori_loop(0, LPAD, z_body, 0)
            for j in range((G + 16) // 16):
                pidx_v[pl.ds(j * 16, 16)] = iota * 0
                lcell_v[pl.ds(j * 16, 16)] = iota * 0 + (LPAD - 1)

            def scan_block(buf, blk, fill):
                def v_body(v, fill):
                    ids16 = buf[pl.ds(v * 16, 16)]
                    match = (ids16 < NCELL) & ((ids16 & (NGROUP - 1)) == gid)
                    csum = plsc.cumsum(match.astype(jnp.int32))
                    pos = fill + csum - 1
                    plsc.store_scatter(lcell_v, [pos],
                                       lax.shift_right_logical(ids16, 7),
                                       mask=match)
                    plsc.store_scatter(pidx_v, [pos],
                                       blk * BS + v * 16 + iota, mask=match)
                    return fill + plsc.all_reduce_population_count(match)[0]

                fill = lax.fori_loop(0, NV, v_body, fill)

                nf = lax.shift_right_logical(fill, 8)

                def drain(j, _):
                    fire(j * G)
                    return 0
                lax.fori_loop(0, nf, drain, 0)

                @pl.when(nf > 0)
                def _():
                    base = nf * G
                    for j in range(G // 16):
                        pidx_v[pl.ds(j * 16, 16)] = pidx_v[pl.ds(base + j * 16, 16)]
                        lcell_v[pl.ds(j * 16, 16)] = lcell_v[pl.ds(base + j * 16, 16)]

                return fill & (G - 1)

            pltpu.async_copy(ids_hbm.at[pl.ds(0, BS)], ids_a, sem2)

            def blk_body(i, fill):
                blk = 2 * i
                pltpu.make_async_copy(ids_hbm.at[pl.ds(0, BS)], ids_a,
                                      sem2).wait()
                pltpu.async_copy(ids_hbm.at[pl.ds((blk + 1) * BS, BS)],
                                 ids_b, sem2)
                fill = scan_block(ids_a, blk, fill)
                pltpu.make_async_copy(ids_hbm.at[pl.ds(0, BS)], ids_b,
                                      sem2).wait()

                @pl.when(blk + 2 < NBLK)
                def _():
                    pltpu.async_copy(ids_hbm.at[pl.ds((blk + 2) * BS, BS)],
                                     ids_a, sem2)

                return scan_block(ids_b, blk + 1, fill)

            fill = lax.fori_loop(0, NBLK // 2, blk_body, 0)

            @pl.when(fill > 0)
            def _():
                for j in range(G // 16):
                    sl = pl.ds(j * 16, 16)
                    tail = (j * 16 + iota) >= fill
                    lcell_v[sl] = jnp.where(tail, LPAD - 1, lcell_v[sl])
                    pidx_v[sl] = jnp.where(tail, 0, pidx_v[sl])
                fire(0)

            for kk in range(LPAD // 128):
                for v in range(8):
                    l16 = kk * 128 + v * 16 + iota
                    widx_v[kk, pl.ds(v * 16, 16)] = l16 * NGROUP + gid
            for kk in range(LPAD // 128):
                pltpu.async_copy(grid_v.at[pl.ds(kk * 128, 128), :],
                                 out_hbm.at[widx_v.at[kk]], sem).wait()
            return 0

        lax.fori_loop(0, NPASS, pass_body, 0)

    return k(ids, feats)


def _bn_relu_kernel(y_ref, mean_ref, rstd_ref, g_ref, be_ref, o_ref):
    mean = mean_ref[0, 0, :]
    rstd = rstd_ref[0, 0, :]
    g = g_ref[0, 0, :]
    be = be_ref[0, 0, :]
    y = y_ref[...]
    yn = (y - mean[None, :, None, None]) * rstd[None, :, None, None]
    yn = yn * g[None, :, None, None] + be[None, :, None, None]
    o_ref[...] = jnp.maximum(yn, 0.0)


def _bn_relu(y, g, be):
    B, C, H, W = y.shape
    mean = jnp.mean(y, axis=(0, 2, 3))
    var = jnp.mean((y - mean[None, :, None, None]) ** 2, axis=(0, 2, 3))
    rstd = jax.lax.rsqrt(var + 1e-5)
    cb = 32
    out = pl.pallas_call(
        _bn_relu_kernel,
        grid=(B, C // cb),
        in_specs=[
            pl.BlockSpec((1, cb, H, W), lambda b, c: (b, c, 0, 0)),
            pl.BlockSpec((1, 1, cb), lambda b, c: (c, 0, 0)),
            pl.BlockSpec((1, 1, cb), lambda b, c: (c, 0, 0)),
            pl.BlockSpec((1, 1, cb), lambda b, c: (c, 0, 0)),
            pl.BlockSpec((1, 1, cb), lambda b, c: (c, 0, 0)),
        ],
        out_specs=pl.BlockSpec((1, cb, H, W), lambda b, c: (b, c, 0, 0)),
        out_shape=jax.ShapeDtypeStruct((B, C, H, W), jnp.float32),
    )(y, mean.reshape(C // cb, 1, cb), rstd.reshape(C // cb, 1, cb),
      g.reshape(C // cb, 1, cb), be.reshape(C // cb, 1, cb))
    return out


def _conv_bn_relu(x, w, b, g, be):
    y = jax.lax.conv_general_dilated(
        x, w, (1, 1), 'SAME', dimension_numbers=('NCHW', 'OIHW', 'NCHW'))
    y = y + b[None, :, None, None]
    return _bn_relu(y, g, be)


def kernel(points, W1, b1, g1, be1, W2, b2, c1w, c1b, bn1g, bn1b, c2w, c2b,
           bn2g, bn2b, c3w, c3b, bn3g, bn3b):
    B, N, _ = points.shape
    px = points[..., 0]
    py = points[..., 1]
    nz = (px != 0) | (py != 0)
    gx = ((px - X0) / VX).astype(jnp.int32)
    gy = ((py - Y0) / VY).astype(jnp.int32)
    valid = nz & (gx >= 0) & (gx < BEV_W) & (gy >= 0) & (gy < BEV_H)
    m = valid.astype(jnp.float32)[..., None]
    f = jnp.einsum('bnd,cd->bnc', points, W1) + b1
    cnt = jnp.maximum(jnp.sum(m, axis=1, keepdims=True), 1.0)
    mean = jnp.sum(f * m, axis=1, keepdims=True) / cnt
    var = jnp.sum(((f - mean) ** 2) * m, axis=1, keepdims=True) / cnt
    f = (f - mean) / jnp.sqrt(var + 1e-5) * g1 + be1
    f = jax.nn.relu(f)
    f = jnp.einsum('bnc,oc->bno', f, W2) + b2
    bidx = jnp.broadcast_to(jnp.arange(B)[:, None], (B, N))
    flat = bidx * (BEV_H * BEV_W) + gy * BEV_W + gx
    flat = jnp.where(valid, flat, SENTINEL)

    grid = _scatter_max_sc(flat.reshape(-1), f.reshape(-1, 128))
    grid = grid[:NCELL].reshape(B, BEV_H, BEV_W, 128).transpose(0, 3, 1, 2)

    h = _conv_bn_relu(grid, c1w, c1b, bn1g, bn1b)
    h = _conv_bn_relu(h, c2w, c2b, bn2g, bn2b)
    h = _conv_bn_relu(h, c3w, c3b, bn3g, bn3b)
    return h


# speedup vs baseline: 1.4667x; 1.1926x over previous
import functools

import jax
import jax.numpy as jnp
from jax import lax
from jax.experimental import pallas as pl
from jax.experimental.pallas import tpu as pltpu
from jax.experimental.pallas import tpu_sc as plsc

VX, VY = 0.512, 0.512
X0, Y0 = -51.2, -51.2
BEV_H, BEV_W = 200, 200
NCELL = 2 * BEV_H * BEV_W
NPTS = 2 * 100000
SENTINEL = 1 << 20

NW = 32
NGROUP = 128
NPASS = NGROUP // NW
LCELLS = NCELL // NGROUP
LPAD = 640
G = 256
BS = 2000
NBLK = NPTS // BS
NV = BS // 16


def _scatter_max_sc(ids, feats):
    mesh = plsc.VectorSubcoreMesh(core_axis_name="c", subcore_axis_name="s")

    @functools.partial(
        pl.kernel,
        mesh=mesh,
        compiler_params=pltpu.CompilerParams(needs_layout_passes=False),
        out_type=jax.ShapeDtypeStruct((LPAD * NGROUP, 128), jnp.float32),
        scratch_types=[
            pltpu.VMEM((LPAD, 128), jnp.float32),
            pltpu.VMEM((BS,), jnp.int32),
            pltpu.VMEM((BS,), jnp.int32),
            pltpu.VMEM((G + BS + 48,), jnp.int32),
            pltpu.VMEM((G + BS + 48,), jnp.int32),
            pltpu.VMEM((G, 128), jnp.float32),
            pltpu.VMEM((LPAD // 128, 128), jnp.int32),
            pltpu.SemaphoreType.DMA,
            pltpu.SemaphoreType.DMA,
        ],
    )
    def k(ids_hbm, feats_hbm, out_hbm, grid_v, ids_a, ids_b, pidx_v, lcell_v,
          rows_v, widx_v, sem, sem2):
        wid = lax.axis_index("s") * 2 + lax.axis_index("c")
        iota = lax.iota(jnp.int32, 16)
        zf = jnp.zeros((16,), jnp.float32)

        def fire(off):
            pltpu.async_copy(feats_hbm.at[pidx_v.at[pl.ds(off, G)]],
                             rows_v, sem).wait()

            def rmw16(j, _):
                lvec = lcell_v[pl.ds(off + j * 16, 16)]
                for t in range(16):
                    l = lvec[t]
                    for r in range(8):
                        cur = grid_v[l, pl.ds(r * 16, 16)]
                        val = rows_v[j * 16 + t, pl.ds(r * 16, 16)]
                        grid_v[l, pl.ds(r * 16, 16)] = jnp.maximum(cur, val)
                return 0

            lax.fori_loop(0, G // 16, rmw16, 0)

        def pass_body(p, _):
            gid = p * NW + wid

            def z_body(i, __):
                for r in range(8):
                    grid_v[i, pl.ds(r * 16, 16)] = zf
                return 0
            lax.fori_loop(0, LPAD, z_body, 0)
            for j in range((G + 16) // 16):
                pidx_v[pl.ds(j * 16, 16)] = iota * 0
                lcell_v[pl.ds(j * 16, 16)] = iota * 0 + (LPAD - 1)

            def scan_block(buf, blk, fill):
                def v_body(v, fill):
                    ids16 = buf[pl.ds(v * 16, 16)]
                    match = (ids16 < NCELL) & ((ids16 & (NGROUP - 1)) == gid)
                    csum = plsc.cumsum(match.astype(jnp.int32))
                    pos = fill + csum - 1
                    plsc.store_scatter(lcell_v, [pos],
                                       lax.shift_right_logical(ids16, 7),
                                       mask=match)
                    plsc.store_scatter(pidx_v, [pos],
                                       blk * BS + v * 16 + iota, mask=match)
                    return fill + plsc.all_reduce_population_count(match)[0]

                fill = lax.fori_loop(0, NV, v_body, fill)

                nf = lax.shift_right_logical(fill, 8)

                def drain(j, _):
                    fire(j * G)
                    return 0
                lax.fori_loop(0, nf, drain, 0)

                @pl.when(nf > 0)
                def _():
                    base = nf * G
                    for j in range(G // 16):
                        pidx_v[pl.ds(j * 16, 16)] = pidx_v[pl.ds(base + j * 16, 16)]
                        lcell_v[pl.ds(j * 16, 16)] = lcell_v[pl.ds(base + j * 16, 16)]

                return fill & (G - 1)

            pltpu.async_copy(ids_hbm.at[pl.ds(0, BS)], ids_a, sem2)

            def blk_body(i, fill):
                blk = 2 * i
                pltpu.make_async_copy(ids_hbm.at[pl.ds(0, BS)], ids_a,
                                      sem2).wait()
                pltpu.async_copy(ids_hbm.at[pl.ds((blk + 1) * BS, BS)],
                                 ids_b, sem2)
                fill = scan_block(ids_a, blk, fill)
                pltpu.make_async_copy(ids_hbm.at[pl.ds(0, BS)], ids_b,
                                      sem2).wait()

                @pl.when(blk + 2 < NBLK)
                def _():
                    pltpu.async_copy(ids_hbm.at[pl.ds((blk + 2) * BS, BS)],
                                     ids_a, sem2)

                return scan_block(ids_b, blk + 1, fill)

            fill = lax.fori_loop(0, NBLK // 2, blk_body, 0)

            @pl.when(fill > 0)
            def _():
                for j in range(G // 16):
                    sl = pl.ds(j * 16, 16)
                    tail = (j * 16 + iota) >= fill
                    lcell_v[sl] = jnp.where(tail, LPAD - 1, lcell_v[sl])
                    pidx_v[sl] = jnp.where(tail, 0, pidx_v[sl])
                fire(0)

            for kk in range(LPAD // 128):
                for v in range(8):
                    l16 = kk * 128 + v * 16 + iota
                    widx_v[kk, pl.ds(v * 16, 16)] = l16 * NGROUP + gid
            for kk in range(LPAD // 128):
                pltpu.async_copy(grid_v.at[pl.ds(kk * 128, 128), :],
                                 out_hbm.at[widx_v.at[kk]], sem).wait()
            return 0

        lax.fori_loop(0, NPASS, pass_body, 0)

    return k(ids, feats)


def _conv_pallas(x, w, bias, s, t, relu_in):
    B, H, W_, Cin = x.shape
    Cout = w.shape[0]
    w9 = jnp.transpose(w, (2, 3, 1, 0)).reshape(9, Cin, Cout)

    def body(x0_ref, x1_ref, x2_ref, w_ref, b_ref, s_ref, t_ref,
             y_ref, st_ref):
        r = pl.program_id(1)
        first = (pl.program_id(0) == 0) & (r == 0)

        @pl.when(first)
        def _():
            st_ref[...] = jnp.zeros_like(st_ref)

        acc = jnp.zeros((W_, Cout), jnp.float32)
        zrow = jnp.zeros((1, Cin), jnp.float32)
        for dy, xr in ((0, x0_ref), (1, x1_ref), (2, x2_ref)):
            xrow = xr[0, 0]
            if relu_in:
                xrow = jnp.maximum(xrow * s_ref[0] + t_ref[0], 0.0)
            ok = jnp.where(dy == 0, r > 0,
                           jnp.where(dy == 2, r < H - 1, True))
            xrow = jnp.where(ok, xrow, 0.0)
            xpad = jnp.concatenate([zrow, xrow, zrow], axis=0)
            for dx in range(3):
                acc = acc + jnp.dot(xpad[dx:dx + W_],
                                    w_ref[dy * 3 + dx],
                                    preferred_element_type=jnp.float32)
        y = acc + b_ref[0]
        y_ref[0, 0] = y
        st_ref[0, :] += jnp.sum(y, axis=0)
        st_ref[1, :] += jnp.sum(y * y, axis=0)

    xmaps = [
        lambda b, r: (b, jnp.maximum(r - 1, 0), 0, 0),
        lambda b, r: (b, r, 0, 0),
        lambda b, r: (b, jnp.minimum(r + 1, H - 1), 0, 0),
    ]
    y, st = pl.pallas_call(
        body,
        grid=(B, H),
        in_specs=[pl.BlockSpec((1, 1, W_, Cin), m) for m in xmaps] + [
            pl.BlockSpec((9, Cin, Cout), lambda b, r: (0, 0, 0)),
            pl.BlockSpec((1, Cout), lambda b, r: (0, 0)),
            pl.BlockSpec((1, Cin), lambda b, r: (0, 0)),
            pl.BlockSpec((1, Cin), lambda b, r: (0, 0)),
        ],
        out_specs=[
            pl.BlockSpec((1, 1, W_, Cout), lambda b, r: (b, r, 0, 0)),
            pl.BlockSpec((2, Cout), lambda b, r: (0, 0)),
        ],
        out_shape=[
            jax.ShapeDtypeStruct((B, H, W_, Cout), jnp.float32),
            jax.ShapeDtypeStruct((2, Cout), jnp.float32),
        ],
    )(x, x, x, w9, bias[None], s[None], t[None])
    return y, st


def _bn_scale(st, n, g, be):
    mean = st[0] / n
    var = st[1] / n - mean * mean
    rstd = jax.lax.rsqrt(var + 1e-5)
    return g * rstd, be - mean * g * rstd


def _bn_relu_nhwc(y, s, t):
    B, H, W_, C = y.shape

    def body(y_ref, s_ref, t_ref, o_ref):
        o_ref[...] = jnp.maximum(y_ref[...] * s_ref[0] + t_ref[0], 0.0)

    rb = 8
    return pl.pallas_call(
        body,
        grid=(B, H // rb),
        in_specs=[
            pl.BlockSpec((1, rb, W_, C), lambda b, r: (b, r, 0, 0)),
            pl.BlockSpec((1, C), lambda b, r: (0, 0)),
            pl.BlockSpec((1, C), lambda b, r: (0, 0)),
        ],
        out_specs=pl.BlockSpec((1, rb, W_, C), lambda b, r: (b, r, 0, 0)),
        out_shape=jax.ShapeDtypeStruct((B, H, W_, C), jnp.float32),
    )(y, s[None], t[None])


def kernel(points, W1, b1, g1, be1, W2, b2, c1w, c1b, bn1g, bn1b, c2w, c2b,
           bn2g, bn2b, c3w, c3b, bn3g, bn3b):
    B, N, _ = points.shape
    px = points[..., 0]
    py = points[..., 1]
    nz = (px != 0) | (py != 0)
    gx = ((px - X0) / VX).astype(jnp.int32)
    gy = ((py - Y0) / VY).astype(jnp.int32)
    valid = nz & (gx >= 0) & (gx < BEV_W) & (gy >= 0) & (gy < BEV_H)
    m = valid.astype(jnp.float32)[..., None]
    f = jnp.einsum('bnd,cd->bnc', points, W1) + b1
    cnt = jnp.maximum(jnp.sum(m, axis=1, keepdims=True), 1.0)
    mean = jnp.sum(f * m, axis=1, keepdims=True) / cnt
    var = jnp.sum(((f - mean) ** 2) * m, axis=1, keepdims=True) / cnt
    f = (f - mean) / jnp.sqrt(var + 1e-5) * g1 + be1
    f = jax.nn.relu(f)
    f = jnp.einsum('bnc,oc->bno', f, W2) + b2
    bidx = jnp.broadcast_to(jnp.arange(B)[:, None], (B, N))
    flat = bidx * (BEV_H * BEV_W) + gy * BEV_W + gx
    flat = jnp.where(valid, flat, SENTINEL)

    grid = _scatter_max_sc(flat.reshape(-1), f.reshape(-1, 128))
    grid = grid[:NCELL].reshape(B, BEV_H, BEV_W, 128)

    n = jnp.float32(B * BEV_H * BEV_W)
    one128 = jnp.ones((128,), jnp.float32)
    zero128 = jnp.zeros((128,), jnp.float32)
    y1, st1 = _conv_pallas(grid, c1w, c1b, one128, zero128, relu_in=False)
    s1, t1 = _bn_scale(st1, n, bn1g, bn1b)
    y2, st2 = _conv_pallas(y1, c2w, c2b, s1, t1, relu_in=True)
    s2, t2 = _bn_scale(st2, n, bn2g, bn2b)
    y3, st3 = _conv_pallas(y2, c3w, c3b, s2, t2, relu_in=True)
    s3, t3 = _bn_scale(st3, n, bn3g, bn3b)
    h = _bn_relu_nhwc(y3, s3, t3)
    return h.transpose(0, 3, 1, 2)
